# baseline (speedup 1.0000x reference)
import functools
import math

import jax
import jax.numpy as jnp
from jax import lax
from jax.experimental import pallas as pl
from jax.experimental.pallas import tpu as pltpu

D = 1024; H = 16; DH = D // H; L = 2; E = 8; DFF = 2048; VOCAB = 8192
B = 1; T = 2048; S = 2048; CF = 1.25
N_TOK = B * T
CAP = int(CF * N_TOK / E)


def _addln_body(a_ref, b_ref, rs_ref, g_ref, bb_ref, x_ref, y_ref):
    x = a_ref[...] + rs_ref[...] * b_ref[...]
    mu = jnp.mean(x, axis=-1, keepdims=True)
    xc = x - mu
    var = jnp.mean(xc * xc, axis=-1, keepdims=True)
    x_ref[...] = x
    y_ref[...] = xc * lax.rsqrt(var + 1e-5) * g_ref[...] + bb_ref[...]


def _add_ln(a, b, rs, g, bb, bt=256):
    grid = (T // bt,)
    return pl.pallas_call(
        _addln_body,
        grid=grid,
        in_specs=[
            pl.BlockSpec((bt, D), lambda i: (i, 0)),
            pl.BlockSpec((bt, D), lambda i: (i, 0)),
            pl.BlockSpec((bt, 1), lambda i: (i, 0)),
            pl.BlockSpec((1, D), lambda i: (0, 0)),
            pl.BlockSpec((1, D), lambda i: (0, 0)),
        ],
        out_specs=[
            pl.BlockSpec((bt, D), lambda i: (i, 0)),
            pl.BlockSpec((bt, D), lambda i: (i, 0)),
        ],
        out_shape=[
            jax.ShapeDtypeStruct((T, D), jnp.float32),
            jax.ShapeDtypeStruct((T, D), jnp.float32),
        ],
    )(a, b, rs, g.reshape(1, D), bb.reshape(1, D))


def _mm_body(x_ref, w_ref, b_ref, o_ref):
    acc = jnp.dot(x_ref[...], w_ref[...], preferred_element_type=jnp.float32)
    o_ref[...] = acc + b_ref[...]


def _matmul(x, w, b, bm=256, bn=512):
    M, K = x.shape
    _, N = w.shape
    bn = min(bn, N)
    bm = min(bm, M)
    return pl.pallas_call(
        _mm_body,
        grid=(M // bm, N // bn),
        in_specs=[
            pl.BlockSpec((bm, K), lambda i, j: (i, 0)),
            pl.BlockSpec((K, bn), lambda i, j: (0, j)),
            pl.BlockSpec((1, bn), lambda i, j: (0, j)),
        ],
        out_specs=pl.BlockSpec((bm, bn), lambda i, j: (i, j)),
        out_shape=jax.ShapeDtypeStruct((M, N), jnp.float32),
    )(x, w, b.reshape(1, N))


def _mm_tn_body(a_ref, b_ref, o_ref):
    o_ref[...] = lax.dot_general(
        a_ref[...], b_ref[...], (((0,), (0,)), ((), ())),
        preferred_element_type=jnp.float32)


def _matmul_tn(a, b, bm=256, bn=512):
    K, M = a.shape
    _, N = b.shape
    return pl.pallas_call(
        _mm_tn_body,
        grid=(M // bm, N // bn),
        in_specs=[
            pl.BlockSpec((K, bm), lambda i, j: (0, i)),
            pl.BlockSpec((K, bn), lambda i, j: (0, j)),
        ],
        out_specs=pl.BlockSpec((bm, bn), lambda i, j: (i, j)),
        out_shape=jax.ShapeDtypeStruct((M, N), jnp.float32),
    )(a, b)


def _emm_body(x_ref, w_ref, b_ref, o_ref, *, relu):
    acc = jnp.dot(x_ref[0], w_ref[0], preferred_element_type=jnp.float32)
    acc = acc + b_ref[0]
    if relu:
        acc = jnp.maximum(acc, 0.0)
    o_ref[0] = acc


def _expert_matmul(x, w, b, relu, bn=512):
    _, M, K = x.shape
    _, _, N = w.shape
    return pl.pallas_call(
        functools.partial(_emm_body, relu=relu),
        grid=(E, N // bn),
        in_specs=[
            pl.BlockSpec((1, M, K), lambda e, j: (e, 0, 0)),
            pl.BlockSpec((1, K, bn), lambda e, j: (e, 0, j)),
            pl.BlockSpec((1, 1, bn), lambda e, j: (e, 0, j)),
        ],
        out_specs=pl.BlockSpec((1, M, bn), lambda e, j: (e, 0, j)),
        out_shape=jax.ShapeDtypeStruct((E, M, N), jnp.float32),
    )(x, w, b.reshape(E, 1, N))


def _attn_body(q_ref, k_ref, v_ref, o_ref, *, causal, bt):
    q = q_ref[0]
    k = k_ref[0]
    s = lax.dot_general(q, k, (((1,), (1,)), ((), ())),
                        preferred_element_type=jnp.float32)
    s = s * (1.0 / math.sqrt(DH))
    if causal:
        i = pl.program_id(1)
        row = i * bt + lax.broadcasted_iota(jnp.int32, (bt, S), 0)
        col = lax.broadcasted_iota(jnp.int32, (bt, S), 1)
        s = jnp.where(col > row, s - 1e9, s)
    m = jnp.max(s, axis=-1, keepdims=True)
    p = jnp.exp(s - m)
    p = p / jnp.sum(p, axis=-1, keepdims=True)
    o_ref[0] = jnp.dot(p, v_ref[0], preferred_element_type=jnp.float32)


def _attention(q, k, v, causal, bt=256):
    return pl.pallas_call(
        functools.partial(_attn_body, causal=causal, bt=bt),
        grid=(H, T // bt),
        in_specs=[
            pl.BlockSpec((1, bt, DH), lambda h, i: (h, i, 0)),
            pl.BlockSpec((1, S, DH), lambda h, i: (h, 0, 0)),
            pl.BlockSpec((1, S, DH), lambda h, i: (h, 0, 0)),
        ],
        out_specs=pl.BlockSpec((1, bt, DH), lambda h, i: (h, i, 0)),
        out_shape=jax.ShapeDtypeStruct((H, T, DH), jnp.float32),
    )(q, k, v)


def _heads(x):
    return x.reshape(T, H, DH).transpose(1, 0, 2)


def _unheads(x):
    return x.transpose(1, 0, 2).reshape(T, D)


def _mha(xln, kv_src, wqkv, bqkv, wo, bo, causal):
    wq, wk, wv = wqkv[:D], wqkv[D:2 * D], wqkv[2 * D:]
    bq, bk, bv = bqkv[:D], bqkv[D:2 * D], bqkv[2 * D:]
    q = _matmul(xln, wq.T, bq)
    k = _matmul(kv_src, wk.T, bk)
    v = _matmul(kv_src, wv.T, bv)
    ctx = _attention(_heads(q), _heads(k), _heads(v), causal)
    return _matmul(_unheads(ctx), wo.T, bo)


def _switch(xln, rw, rb, w1, b1, w2, b2):
    logits = _matmul(xln, rw, rb, bm=256, bn=E)
    zmax = jnp.max(logits, axis=-1)
    z = zmax + jnp.log(jnp.sum(jnp.exp(logits - zmax[:, None]), axis=-1))
    z_loss = jnp.mean(z * z)
    probs = jax.nn.softmax(logits, axis=-1)
    eidx = jnp.argmax(probs, axis=-1)
    gate = jnp.max(probs, axis=-1)
    onehot = jax.nn.one_hot(eidx, E, dtype=jnp.float32)
    f = jnp.mean(onehot, axis=0)
    p = jnp.mean(probs, axis=0)
    lb_loss = E * jnp.sum(f * p)
    pos = jnp.cumsum(onehot, axis=0) * onehot
    slot = jnp.sum(pos, axis=-1).astype(jnp.int32) - 1
    keep = ((slot >= 0) & (slot < CAP)).astype(jnp.float32)
    col = eidx.astype(jnp.int32) * CAP + jnp.clip(slot, 0, CAP - 1)
    disp = jax.nn.one_hot(col, E * CAP, dtype=jnp.float32) * keep[:, None]
    einp = _matmul_tn(disp, xln)
    hid = _expert_matmul(einp.reshape(E, CAP, D), w1, b1, relu=True)
    eout = _expert_matmul(hid, w2, b2, relu=False)
    y = _matmul(disp * gate[:, None], eout.reshape(E * CAP, D), jnp.zeros((D,), jnp.float32), bm=256, bn=512)
    return y, lb_loss, z_loss


def _make_pe():
    import numpy as np
    pos = np.arange(T)[:, None].astype(np.float32)
    i = np.arange(0, D, 2).astype(np.float32)[None, :]
    ang = pos / np.power(10000.0, i / D)
    pe = np.zeros((T, D), dtype=np.float32)
    pe[:, 0::2] = np.sin(ang)
    pe[:, 1::2] = np.cos(ang)
    return jnp.asarray(pe)


_PE = _make_pe()


def kernel(tgt, src, tgt_mask, tgt_pad_mask, src_pad_mask, emb,
           ln1_g, ln1_b, ln2_g, ln2_b, ln3_g, ln3_b,
           self_wqkv, self_bqkv, self_wo, self_bo,
           cross_wqkv, cross_bqkv, cross_wo, cross_bo,
           router_w, router_b, ew1, eb1, ew2, eb2,
           end_g, end_b, fc_w, fc_b):
    del tgt_mask, tgt_pad_mask, src_pad_mask
    emb_g = emb[tgt[0]]
    src2 = src[0]
    ones_rs = jnp.ones((T, 1), jnp.float32)
    sqrt_rs = jnp.full((T, 1), math.sqrt(float(D)), jnp.float32)
    pe = _PE

    lb_sum = jnp.float32(0.0)
    z_sum = jnp.float32(0.0)
    x, xln = _add_ln(pe, emb_g, sqrt_rs, ln1_g[0], ln1_b[0])
    for l in range(L):
        h = _mha(xln, xln, self_wqkv[l], self_bqkv[l], self_wo[l], self_bo[l], causal=True)
        x, xln = _add_ln(x, h, ones_rs, ln2_g[l], ln2_b[l])
        h = _mha(xln, src2, cross_wqkv[l], cross_bqkv[l], cross_wo[l], cross_bo[l], causal=False)
        x, xln = _add_ln(x, h, ones_rs, ln3_g[l], ln3_b[l])
        y, lb, zl = _switch(xln, router_w[l], router_b[l], ew1[l], eb1[l], ew2[l], eb2[l])
        lb_sum = lb_sum + lb
        z_sum = z_sum + zl
        if l + 1 < L:
            x, xln = _add_ln(x, y, ones_rs, ln1_g[l + 1], ln1_b[l + 1])
        else:
            x, xln = _add_ln(x, y, ones_rs, end_g, end_b)
    out = _matmul(xln, fc_w.T, fc_b, bm=256, bn=512)
    return out.reshape(B, T, VOCAB), lb_sum / L, z_sum / L


# speedup vs baseline: 1.1015x; 1.1015x over previous
import functools
import math

import jax
import jax.numpy as jnp
from jax import lax
from jax.experimental import pallas as pl
from jax.experimental.pallas import tpu as pltpu

D = 1024; H = 16; DH = D // H; L = 2; E = 8; DFF = 2048; VOCAB = 8192
B = 1; T = 2048; S = 2048; CF = 1.25
N_TOK = B * T
CAP = int(CF * N_TOK / E)


def _addln_body(a_ref, b_ref, rs_ref, g_ref, bb_ref, x_ref, y_ref):
    x = a_ref[...] + rs_ref[...] * b_ref[...]
    mu = jnp.mean(x, axis=-1, keepdims=True)
    xc = x - mu
    var = jnp.mean(xc * xc, axis=-1, keepdims=True)
    x_ref[...] = x
    y_ref[...] = xc * lax.rsqrt(var + 1e-5) * g_ref[...] + bb_ref[...]


def _add_ln(a, b, rs, g, bb, bt=256):
    grid = (T // bt,)
    return pl.pallas_call(
        _addln_body,
        grid=grid,
        in_specs=[
            pl.BlockSpec((bt, D), lambda i: (i, 0)),
            pl.BlockSpec((bt, D), lambda i: (i, 0)),
            pl.BlockSpec((bt, 1), lambda i: (i, 0)),
            pl.BlockSpec((1, D), lambda i: (0, 0)),
            pl.BlockSpec((1, D), lambda i: (0, 0)),
        ],
        out_specs=[
            pl.BlockSpec((bt, D), lambda i: (i, 0)),
            pl.BlockSpec((bt, D), lambda i: (i, 0)),
        ],
        out_shape=[
            jax.ShapeDtypeStruct((T, D), jnp.float32),
            jax.ShapeDtypeStruct((T, D), jnp.float32),
        ],
    )(a, b, rs, g.reshape(1, D), bb.reshape(1, D))


def _mm_body(x_ref, w_ref, b_ref, o_ref):
    acc = jnp.dot(x_ref[...], w_ref[...], preferred_element_type=jnp.float32)
    o_ref[...] = acc + b_ref[...]


def _matmul(x, w, b, bm=256, bn=512, dtype=jnp.bfloat16):
    x = x.astype(dtype)
    w = w.astype(dtype)
    M, K = x.shape
    _, N = w.shape
    bn = min(bn, N)
    bm = min(bm, M)
    return pl.pallas_call(
        _mm_body,
        grid=(M // bm, N // bn),
        in_specs=[
            pl.BlockSpec((bm, K), lambda i, j: (i, 0)),
            pl.BlockSpec((K, bn), lambda i, j: (0, j)),
            pl.BlockSpec((1, bn), lambda i, j: (0, j)),
        ],
        out_specs=pl.BlockSpec((bm, bn), lambda i, j: (i, j)),
        out_shape=jax.ShapeDtypeStruct((M, N), jnp.float32),
    )(x, w, b.reshape(1, N))


def _mm_tn_body(a_ref, b_ref, o_ref):
    o_ref[...] = lax.dot_general(
        a_ref[...], b_ref[...], (((0,), (0,)), ((), ())),
        preferred_element_type=jnp.float32)


def _matmul_tn(a, b, bm=256, bn=512):
    a = a.astype(jnp.bfloat16)
    b = b.astype(jnp.bfloat16)
    K, M = a.shape
    _, N = b.shape
    return pl.pallas_call(
        _mm_tn_body,
        grid=(M // bm, N // bn),
        in_specs=[
            pl.BlockSpec((K, bm), lambda i, j: (0, i)),
            pl.BlockSpec((K, bn), lambda i, j: (0, j)),
        ],
        out_specs=pl.BlockSpec((bm, bn), lambda i, j: (i, j)),
        out_shape=jax.ShapeDtypeStruct((M, N), jnp.float32),
    )(a, b)


def _emm_body(x_ref, w_ref, b_ref, o_ref, *, relu):
    acc = jnp.dot(x_ref[0], w_ref[0], preferred_element_type=jnp.float32)
    acc = acc + b_ref[0]
    if relu:
        acc = jnp.maximum(acc, 0.0)
    o_ref[0] = acc


def _expert_matmul(x, w, b, relu, bn=512):
    x = x.astype(jnp.bfloat16)
    w = w.astype(jnp.bfloat16)
    _, M, K = x.shape
    _, _, N = w.shape
    return pl.pallas_call(
        functools.partial(_emm_body, relu=relu),
        grid=(E, N // bn),
        in_specs=[
            pl.BlockSpec((1, M, K), lambda e, j: (e, 0, 0)),
            pl.BlockSpec((1, K, bn), lambda e, j: (e, 0, j)),
            pl.BlockSpec((1, 1, bn), lambda e, j: (e, 0, j)),
        ],
        out_specs=pl.BlockSpec((1, M, bn), lambda e, j: (e, 0, j)),
        out_shape=jax.ShapeDtypeStruct((E, M, N), jnp.float32),
    )(x, w, b.reshape(E, 1, N))


def _attn_body(q_ref, k_ref, v_ref, o_ref, *, causal, bt):
    q = q_ref[0]
    k = k_ref[0]
    s = lax.dot_general(q, k, (((1,), (1,)), ((), ())),
                        preferred_element_type=jnp.float32)
    s = s * (1.0 / math.sqrt(DH))
    if causal:
        i = pl.program_id(1)
        row = i * bt + lax.broadcasted_iota(jnp.int32, (bt, S), 0)
        col = lax.broadcasted_iota(jnp.int32, (bt, S), 1)
        s = jnp.where(col > row, s - 1e9, s)
    m = jnp.max(s, axis=-1, keepdims=True)
    p = jnp.exp(s - m)
    p = p / jnp.sum(p, axis=-1, keepdims=True)
    o_ref[0] = jnp.dot(p.astype(jnp.bfloat16), v_ref[0],
                       preferred_element_type=jnp.float32)


def _attention(q, k, v, causal, bt=256):
    q = q.astype(jnp.bfloat16)
    k = k.astype(jnp.bfloat16)
    v = v.astype(jnp.bfloat16)
    return pl.pallas_call(
        functools.partial(_attn_body, causal=causal, bt=bt),
        grid=(H, T // bt),
        in_specs=[
            pl.BlockSpec((1, bt, DH), lambda h, i: (h, i, 0)),
            pl.BlockSpec((1, S, DH), lambda h, i: (h, 0, 0)),
            pl.BlockSpec((1, S, DH), lambda h, i: (h, 0, 0)),
        ],
        out_specs=pl.BlockSpec((1, bt, DH), lambda h, i: (h, i, 0)),
        out_shape=jax.ShapeDtypeStruct((H, T, DH), jnp.float32),
    )(q, k, v)


def _heads(x):
    return x.reshape(T, H, DH).transpose(1, 0, 2)


def _unheads(x):
    return x.transpose(1, 0, 2).reshape(T, D)


def _mha(xln, kv_src, wqkv, bqkv, wo, bo, causal):
    wq, wk, wv = wqkv[:D], wqkv[D:2 * D], wqkv[2 * D:]
    bq, bk, bv = bqkv[:D], bqkv[D:2 * D], bqkv[2 * D:]
    q = _matmul(xln, wq.T, bq)
    k = _matmul(kv_src, wk.T, bk)
    v = _matmul(kv_src, wv.T, bv)
    ctx = _attention(_heads(q), _heads(k), _heads(v), causal)
    return _matmul(_unheads(ctx), wo.T, bo)


def _switch(xln, rw, rb, w1, b1, w2, b2):
    logits = _matmul(xln, rw, rb, bm=256, bn=E, dtype=jnp.float32)
    zmax = jnp.max(logits, axis=-1)
    z = zmax + jnp.log(jnp.sum(jnp.exp(logits - zmax[:, None]), axis=-1))
    z_loss = jnp.mean(z * z)
    probs = jax.nn.softmax(logits, axis=-1)
    eidx = jnp.argmax(probs, axis=-1)
    gate = jnp.max(probs, axis=-1)
    onehot = jax.nn.one_hot(eidx, E, dtype=jnp.float32)
    f = jnp.mean(onehot, axis=0)
    p = jnp.mean(probs, axis=0)
    lb_loss = E * jnp.sum(f * p)
    pos = jnp.cumsum(onehot, axis=0) * onehot
    slot = jnp.sum(pos, axis=-1).astype(jnp.int32) - 1
    keep = ((slot >= 0) & (slot < CAP)).astype(jnp.float32)
    col = eidx.astype(jnp.int32) * CAP + jnp.clip(slot, 0, CAP - 1)
    disp = jax.nn.one_hot(col, E * CAP, dtype=jnp.float32) * keep[:, None]
    einp = _matmul_tn(disp, xln)
    hid = _expert_matmul(einp.reshape(E, CAP, D), w1, b1, relu=True)
    eout = _expert_matmul(hid, w2, b2, relu=False)
    y = _matmul(disp * gate[:, None], eout.reshape(E * CAP, D), jnp.zeros((D,), jnp.float32), bm=256, bn=512)
    return y, lb_loss, z_loss


def _make_pe():
    import numpy as np
    pos = np.arange(T)[:, None].astype(np.float32)
    i = np.arange(0, D, 2).astype(np.float32)[None, :]
    ang = pos / np.power(10000.0, i / D)
    pe = np.zeros((T, D), dtype=np.float32)
    pe[:, 0::2] = np.sin(ang)
    pe[:, 1::2] = np.cos(ang)
    return jnp.asarray(pe)


_PE = _make_pe()


def kernel(tgt, src, tgt_mask, tgt_pad_mask, src_pad_mask, emb,
           ln1_g, ln1_b, ln2_g, ln2_b, ln3_g, ln3_b,
           self_wqkv, self_bqkv, self_wo, self_bo,
           cross_wqkv, cross_bqkv, cross_wo, cross_bo,
           router_w, router_b, ew1, eb1, ew2, eb2,
           end_g, end_b, fc_w, fc_b):
    del tgt_mask, tgt_pad_mask, src_pad_mask
    emb_g = emb[tgt[0]]
    src2 = src[0]
    ones_rs = jnp.ones((T, 1), jnp.float32)
    sqrt_rs = jnp.full((T, 1), math.sqrt(float(D)), jnp.float32)
    pe = _PE

    lb_sum = jnp.float32(0.0)
    z_sum = jnp.float32(0.0)
    x, xln = _add_ln(pe, emb_g, sqrt_rs, ln1_g[0], ln1_b[0])
    for l in range(L):
        h = _mha(xln, xln, self_wqkv[l], self_bqkv[l], self_wo[l], self_bo[l], causal=True)
        x, xln = _add_ln(x, h, ones_rs, ln2_g[l], ln2_b[l])
        h = _mha(xln, src2, cross_wqkv[l], cross_bqkv[l], cross_wo[l], cross_bo[l], causal=False)
        x, xln = _add_ln(x, h, ones_rs, ln3_g[l], ln3_b[l])
        y, lb, zl = _switch(xln, router_w[l], router_b[l], ew1[l], eb1[l], ew2[l], eb2[l])
        lb_sum = lb_sum + lb
        z_sum = z_sum + zl
        if l + 1 < L:
            x, xln = _add_ln(x, y, ones_rs, ln1_g[l + 1], ln1_b[l + 1])
        else:
            x, xln = _add_ln(x, y, ones_rs, end_g, end_b)
    out = _matmul(xln, fc_w.T, fc_b, bm=256, bn=512)
    return out.reshape(B, T, VOCAB), lb_sum / L, z_sum / L
